# baseline (speedup 1.0000x reference)
import functools

import jax
import jax.numpy as jnp
from jax.experimental import pallas as pl
from jax.experimental.pallas import tpu as pltpu

IOU_THR = 0.4
BI = 512
BJ = 512
BIG = 1e30


def _nms_decay_body(c0_ref, c1_ref,
                    x1i_ref, y1i_ref, x2i_ref, y2i_ref, si_ref,
                    x1j_ref, y1j_ref, x2j_ref, y2j_ref, sj_ref,
                    out_ref):
    b = pl.program_id(0)
    lo = c0_ref[b]
    hi = c1_ref[b]

    x1i = x1i_ref[...]
    y1i = y1i_ref[...]
    x2i = x2i_ref[...]
    y2i = y2i_ref[...]
    si = si_ref[...]
    area_i = (x2i - x1i) * (y2i - y1i)

    def body(c, acc):
        sl = pl.ds(c * BJ, BJ)
        x1j = x1j_ref[:, sl]
        y1j = y1j_ref[:, sl]
        x2j = x2j_ref[:, sl]
        y2j = y2j_ref[:, sl]
        sj = sj_ref[:, sl]
        area_j = (x2j - x1j) * (y2j - y1j)

        w = jnp.maximum(jnp.minimum(x2i, x2j) - jnp.maximum(x1i, x1j), 0.0)
        h = jnp.maximum(jnp.minimum(y2i, y2j) - jnp.maximum(y1i, y1j), 0.0)
        inter = w * h
        union = (area_i + area_j) - inter
        iou = inter / union
        cond = jnp.logical_and(iou > IOU_THR, sj > si)
        f = jnp.where(cond, 1.0 - iou, 1.0)
        return acc * f

    acc = jax.lax.fori_loop(lo, hi, body,
                            jnp.ones((BI, BJ), jnp.float32))

    width = BJ
    while width > 1:
        width //= 2
        acc = acc[:, :width] * acc[:, width:2 * width]

    out_ref[...] = si * acc


@jax.jit
def kernel(boxes, scores):
    n = boxes.shape[0]
    npad = ((n + BI - 1) // BI) * BI
    pad = npad - n

    x1 = boxes[:, 0]
    y1 = boxes[:, 1]
    x2p = boxes[:, 2] + 1.0
    y2p = boxes[:, 3] + 1.0
    maxext = jnp.maximum(jnp.max(x2p - x1), jnp.max(y2p - y1))

    order = jnp.argsort(x1)
    xs1 = jnp.pad(x1[order], (0, pad), constant_values=BIG)
    ys1 = jnp.pad(y1[order], (0, pad), constant_values=BIG)
    xs2 = jnp.pad(x2p[order], (0, pad), constant_values=BIG)
    ys2 = jnp.pad(y2p[order], (0, pad), constant_values=BIG)
    ss = jnp.pad(scores[order], (0, pad), constant_values=-BIG)

    nb = npad // BI
    blk = xs1.reshape(nb, BI)
    lo_idx = jnp.searchsorted(xs1, blk[:, 0] - maxext, side='left')
    hi_idx = jnp.searchsorted(xs1, blk[:, -1] + maxext, side='right')
    c0 = (lo_idx // BJ).astype(jnp.int32)
    c1 = ((hi_idx + BJ - 1) // BJ).astype(jnp.int32)

    col = lambda a: a.reshape(npad, 1)
    row = lambda a: a.reshape(1, npad)

    ispec = pl.BlockSpec((BI, 1), lambda i: (i, 0))
    jspec = pl.BlockSpec((1, npad), lambda i: (0, 0))
    sspec = pl.BlockSpec(memory_space=pltpu.SMEM)

    out = pl.pallas_call(
        _nms_decay_body,
        grid=(nb,),
        in_specs=[sspec, sspec,
                  ispec, ispec, ispec, ispec, ispec,
                  jspec, jspec, jspec, jspec, jspec],
        out_specs=pl.BlockSpec((BI, 1), lambda i: (i, 0)),
        out_shape=jax.ShapeDtypeStruct((npad, 1), jnp.float32),
    )(c0, c1,
      col(xs1), col(ys1), col(xs2), col(ys2), col(ss),
      row(xs1), row(ys1), row(xs2), row(ys2), row(ss))

    decayed_sorted = out[:n, 0]
    return jnp.zeros((n,), jnp.float32).at[order].set(decayed_sorted)


# speedup vs baseline: 1.6793x; 1.6793x over previous
import functools

import jax
import jax.numpy as jnp
from jax.experimental import pallas as pl
from jax.experimental.pallas import tpu as pltpu

IOU_THR = 0.4
BI = 512
BJ = 512
BIG = 1e30


def _nms_decay_body(c0_ref, c1_ref,
                    x1i_ref, y1i_ref, x2i_ref, y2i_ref, si_ref,
                    x1j_ref, y1j_ref, x2j_ref, y2j_ref, sj_ref,
                    out_ref):
    b = pl.program_id(0)
    lo = c0_ref[b]
    hi = c1_ref[b]

    x1i = x1i_ref[...]
    y1i = y1i_ref[...]
    x2i = x2i_ref[...]
    y2i = y2i_ref[...]
    si = si_ref[...]
    area_i = (x2i - x1i) * (y2i - y1i)

    def body(c, acc):
        sl = pl.ds(c * BJ, BJ)
        x1j = x1j_ref[:, sl]
        y1j = y1j_ref[:, sl]
        x2j = x2j_ref[:, sl]
        y2j = y2j_ref[:, sl]
        sj = sj_ref[:, sl]
        area_j = (x2j - x1j) * (y2j - y1j)

        w = jnp.maximum(jnp.minimum(x2i, x2j) - jnp.maximum(x1i, x1j), 0.0)
        h = jnp.maximum(jnp.minimum(y2i, y2j) - jnp.maximum(y1i, y1j), 0.0)
        inter = w * h
        union = (area_i + area_j) - inter
        iou = inter / union
        cond = jnp.logical_and(iou > IOU_THR, sj > si)
        f = jnp.where(cond, 1.0 - iou, 1.0)
        return acc * f

    acc = jax.lax.fori_loop(lo, hi, body,
                            jnp.ones((BI, BJ), jnp.float32))

    width = BJ
    while width > 1:
        width //= 2
        acc = acc[:, :width] * acc[:, width:2 * width]

    out_ref[...] = si * acc


@jax.jit
def kernel(boxes, scores):
    n = boxes.shape[0]
    npad = ((n + BI - 1) // BI) * BI
    pad = npad - n

    x1 = boxes[:, 0]
    y1 = boxes[:, 1]
    x2p = boxes[:, 2] + 1.0
    y2p = boxes[:, 3] + 1.0
    maxext = jnp.maximum(jnp.max(x2p - x1), jnp.max(y2p - y1))

    iota = jnp.arange(n, dtype=jnp.int32)
    xs1, ys1, xs2, ys2, ss, order = jax.lax.sort(
        (x1, y1, x2p, y2p, scores, iota), num_keys=1)
    xs1 = jnp.pad(xs1, (0, pad), constant_values=BIG)
    ys1 = jnp.pad(ys1, (0, pad), constant_values=BIG)
    xs2 = jnp.pad(xs2, (0, pad), constant_values=BIG)
    ys2 = jnp.pad(ys2, (0, pad), constant_values=BIG)
    ss = jnp.pad(ss, (0, pad), constant_values=-BIG)

    nb = npad // BI
    blk = xs1.reshape(nb, BI)
    lo_idx = jnp.searchsorted(xs1, blk[:, 0] - maxext, side='left')
    hi_idx = jnp.searchsorted(xs1, blk[:, -1] + maxext, side='right')
    c0 = (lo_idx // BJ).astype(jnp.int32)
    c1 = ((hi_idx + BJ - 1) // BJ).astype(jnp.int32)

    col = lambda a: a.reshape(npad, 1)
    row = lambda a: a.reshape(1, npad)

    ispec = pl.BlockSpec((BI, 1), lambda i: (i, 0))
    jspec = pl.BlockSpec((1, npad), lambda i: (0, 0))
    sspec = pl.BlockSpec(memory_space=pltpu.SMEM)

    out = pl.pallas_call(
        _nms_decay_body,
        grid=(nb,),
        in_specs=[sspec, sspec,
                  ispec, ispec, ispec, ispec, ispec,
                  jspec, jspec, jspec, jspec, jspec],
        out_specs=pl.BlockSpec((BI, 1), lambda i: (i, 0)),
        out_shape=jax.ShapeDtypeStruct((npad, 1), jnp.float32),
    )(c0, c1,
      col(xs1), col(ys1), col(xs2), col(ys2), col(ss),
      row(xs1), row(ys1), row(xs2), row(ys2), row(ss))

    decayed_sorted = out[:n, 0]
    return jnp.zeros((n,), jnp.float32).at[order].set(decayed_sorted)
